# baseline (speedup 1.0000x reference)
import jax
import jax.numpy as jnp
from jax import lax
from jax.experimental import pallas as pl
from jax.experimental.pallas import tpu as pltpu
from jax.experimental.pallas import tpu_sc as plsc

VOCAB = 151936
EMBED_DIM = 1024
BATCH = 4
SEQ = 4096
TOTAL = BATCH * SEQ

_INFO = plsc.get_sparse_core_info()
_NC, _NS = _INFO.num_cores, _INFO.num_subcores
_NW = _NC * _NS
_PER_W = TOTAL // _NW
_CHUNK = 16
_NCHUNK = _PER_W // _CHUNK
_NBUF = 4
_W_PER_ROW = SEQ // _PER_W


def _embed_body(ids_hbm, table_hbm, out_hbm, idx_v, b0, b1, b2, b3,
                g0, g1, g2, g3, w0, w1, w2, w3):
    bufs = (b0, b1, b2, b3)
    gsems = (g0, g1, g2, g3)
    wsems = (w0, w1, w2, w3)
    wid = lax.axis_index("s") * _NC + lax.axis_index("c")
    brow = wid // _W_PER_ROW
    col = (wid % _W_PER_ROW) * _PER_W
    pltpu.sync_copy(ids_hbm.at[brow, pl.ds(col, _PER_W)], idx_v)

    def gather(c, b):
        return pltpu.async_copy(
            table_hbm.at[idx_v.at[pl.ds(c * _CHUNK, _CHUNK)]], bufs[b],
            gsems[b])

    def write(c, b):
        return pltpu.async_copy(
            bufs[b], out_hbm.at[brow, pl.ds(col + c * _CHUNK, _CHUNK)],
            wsems[b])

    def wait_gather(b):
        pltpu.make_async_copy(
            table_hbm.at[idx_v.at[pl.ds(0, _CHUNK)]], bufs[b],
            gsems[b]).wait()

    def wait_write(b):
        pltpu.make_async_copy(
            bufs[b], out_hbm.at[brow, pl.ds(col, _CHUNK)], wsems[b]).wait()

    for b in range(_NBUF):
        gather(b, b)

    @pl.loop(0, _NCHUNK - _NBUF, step=_NBUF)
    def _ring(c0):
        for b in range(_NBUF):
            c = c0 + b
            wait_gather(b)
            write(c, b)
            wait_write(b)
            gather(c + _NBUF, b)

    for b in range(_NBUF):
        wait_gather(b)
        write(_NCHUNK - _NBUF + b, b)
    for b in range(_NBUF):
        wait_write(b)


_embed_call = pl.kernel(
    _embed_body,
    out_type=jax.ShapeDtypeStruct((BATCH, SEQ, EMBED_DIM), jnp.float32),
    mesh=plsc.VectorSubcoreMesh(core_axis_name="c", subcore_axis_name="s"),
    scratch_types=[
        pltpu.VMEM((_PER_W,), jnp.int32),
        pltpu.VMEM((_CHUNK, EMBED_DIM), jnp.float32),
        pltpu.VMEM((_CHUNK, EMBED_DIM), jnp.float32),
        pltpu.VMEM((_CHUNK, EMBED_DIM), jnp.float32),
        pltpu.VMEM((_CHUNK, EMBED_DIM), jnp.float32),
        pltpu.SemaphoreType.DMA,
        pltpu.SemaphoreType.DMA,
        pltpu.SemaphoreType.DMA,
        pltpu.SemaphoreType.DMA,
        pltpu.SemaphoreType.DMA,
        pltpu.SemaphoreType.DMA,
        pltpu.SemaphoreType.DMA,
        pltpu.SemaphoreType.DMA,
    ],
)


@jax.jit
def kernel(input_ids, embed_table):
    return _embed_call(input_ids.astype(jnp.int32), embed_table)


# speedup vs baseline: 1.0080x; 1.0080x over previous
import jax
import jax.numpy as jnp
from jax import lax
from jax.experimental import pallas as pl
from jax.experimental.pallas import tpu as pltpu
from jax.experimental.pallas import tpu_sc as plsc

VOCAB = 151936
EMBED_DIM = 1024
BATCH = 4
SEQ = 4096
TOTAL = BATCH * SEQ

_INFO = plsc.get_sparse_core_info()
_NC, _NS = _INFO.num_cores, _INFO.num_subcores
_NW = _NC * _NS
_PER_W = TOTAL // _NW
_CHUNK = 8
_NCHUNK = _PER_W // _CHUNK
_NBUF = 8
_W_PER_ROW = SEQ // _PER_W


def _embed_body(ids_hbm, table_hbm, out_hbm, idx_v,
                b0, b1, b2, b3, b4, b5, b6, b7,
                g0, g1, g2, g3, g4, g5, g6, g7,
                w0, w1, w2, w3, w4, w5, w6, w7):
    bufs = (b0, b1, b2, b3, b4, b5, b6, b7)
    gsems = (g0, g1, g2, g3, g4, g5, g6, g7)
    wsems = (w0, w1, w2, w3, w4, w5, w6, w7)
    wid = lax.axis_index("s") * _NC + lax.axis_index("c")
    brow = wid // _W_PER_ROW
    col = (wid % _W_PER_ROW) * _PER_W
    pltpu.sync_copy(ids_hbm.at[brow, pl.ds(col, _PER_W)], idx_v)

    def gather(c, b):
        return pltpu.async_copy(
            table_hbm.at[idx_v.at[pl.ds(c * _CHUNK, _CHUNK)]], bufs[b],
            gsems[b])

    def write(c, b):
        return pltpu.async_copy(
            bufs[b], out_hbm.at[brow, pl.ds(col + c * _CHUNK, _CHUNK)],
            wsems[b])

    def wait_gather(b):
        pltpu.make_async_copy(
            table_hbm.at[idx_v.at[pl.ds(0, _CHUNK)]], bufs[b],
            gsems[b]).wait()

    def wait_write(b):
        pltpu.make_async_copy(
            bufs[b], out_hbm.at[brow, pl.ds(col, _CHUNK)], wsems[b]).wait()

    for b in range(_NBUF):
        gather(b, b)

    @pl.loop(0, _NCHUNK - _NBUF, step=_NBUF)
    def _ring(c0):
        for b in range(_NBUF):
            c = c0 + b
            wait_gather(b)
            write(c, b)
            wait_write(b)
            gather(c + _NBUF, b)

    for b in range(_NBUF):
        wait_gather(b)
        write(_NCHUNK - _NBUF + b, b)
    for b in range(_NBUF):
        wait_write(b)


_embed_call = pl.kernel(
    _embed_body,
    out_type=jax.ShapeDtypeStruct((BATCH, SEQ, EMBED_DIM), jnp.float32),
    mesh=plsc.VectorSubcoreMesh(core_axis_name="c", subcore_axis_name="s"),
    scratch_types=[
        pltpu.VMEM((_PER_W,), jnp.int32),
        pltpu.VMEM((_CHUNK, EMBED_DIM), jnp.float32),
        pltpu.VMEM((_CHUNK, EMBED_DIM), jnp.float32),
        pltpu.VMEM((_CHUNK, EMBED_DIM), jnp.float32),
        pltpu.VMEM((_CHUNK, EMBED_DIM), jnp.float32),
        pltpu.VMEM((_CHUNK, EMBED_DIM), jnp.float32),
        pltpu.VMEM((_CHUNK, EMBED_DIM), jnp.float32),
        pltpu.VMEM((_CHUNK, EMBED_DIM), jnp.float32),
        pltpu.VMEM((_CHUNK, EMBED_DIM), jnp.float32),
    ] + [pltpu.SemaphoreType.DMA] * 16,
)


@jax.jit
def kernel(input_ids, embed_table):
    return _embed_call(input_ids.astype(jnp.int32), embed_table)
